# baseline (speedup 1.0000x reference)
import functools

import jax
import jax.numpy as jnp
from jax.experimental import pallas as pl
from jax.experimental.pallas import tpu as pltpu

_NCH = 85
_COLS = 4096


def _decode_kernel(img_ref, in_ref, eye_ref, out_ref, *, cols):
    img_size = img_ref[0]
    a = pl.program_id(0) % 3
    aw = jnp.where(a == 1, 16.0, jnp.where(a == 2, 33.0, 10.0))
    ah = jnp.where(a == 1, 30.0, jnp.where(a == 2, 23.0, 13.0))
    v = in_ref[0]
    sg = 1.0 / (1.0 + jnp.exp(-v))
    ex = jnp.exp(v[0:4, :])
    l = jnp.clip(ex[0:1] * aw, 0.0, img_size)
    t = jnp.clip(ex[1:2] * ah, 0.0, img_size)
    r = jnp.clip(ex[2:3] * aw, 0.0, img_size)
    b = jnp.clip(ex[3:4] * ah, 0.0, img_size)
    k = jax.lax.broadcasted_iota(jnp.int32, (1, cols), 1)
    p = pl.program_id(1) * cols + k
    gx = (p % 128).astype(jnp.float32)
    gy = (p // 128).astype(jnp.float32)
    cx = (gx + 0.5) * 4.0 + (r - l) * 0.5
    cy = (gy + 0.5) * 4.0 + (b - t) * 0.5
    w = l + r
    h = t + b
    cm = jnp.concatenate([cx, cy, w, h, sg[4:, :]], axis=0)
    out_ref[0] = jax.lax.dot_general(
        cm, eye_ref[...], (((0,), (0,)), ((), ())),
        preferred_element_type=jnp.float32)


def kernel(raw, img_size):
    nB = raw.shape[0]
    nG = raw.shape[2]
    nA = 3
    nCH = _NCH
    cols = _COLS
    inp = raw.reshape(nB * nA, nCH, nG * nG)
    eye = jnp.eye(nCH, dtype=jnp.float32)
    img = jnp.asarray(img_size, dtype=jnp.float32).reshape(1)
    grid = (nB * nA, (nG * nG) // cols)
    out = pl.pallas_call(
        functools.partial(_decode_kernel, cols=cols),
        grid=grid,
        in_specs=[
            pl.BlockSpec(memory_space=pltpu.SMEM),
            pl.BlockSpec((1, nCH, cols), lambda i, j: (i, 0, j)),
            pl.BlockSpec((nCH, nCH), lambda i, j: (0, 0)),
        ],
        out_specs=pl.BlockSpec((1, cols, nCH), lambda i, j: (i, j, 0)),
        out_shape=jax.ShapeDtypeStruct((nB * nA, nG * nG, nCH), jnp.float32),
    )(img, inp, eye)
    return out.reshape(nB, nA * nG * nG, nCH)


# speedup vs baseline: 1.0857x; 1.0857x over previous
import functools

import jax
import jax.numpy as jnp
from jax.experimental import pallas as pl
from jax.experimental.pallas import tpu as pltpu

_NCH = 85
_COLS = 16384


def _decode_kernel(img_ref, in_ref, eye_ref, out_ref, *, cols):
    img_size = img_ref[0]
    a = pl.program_id(0) % 3
    aw = jnp.where(a == 1, 16.0, jnp.where(a == 2, 33.0, 10.0))
    ah = jnp.where(a == 1, 30.0, jnp.where(a == 2, 23.0, 13.0))
    v = in_ref[0]
    sg = 1.0 / (1.0 + jnp.exp(-v))
    ex = jnp.exp(v[0:4, :])
    l = jnp.clip(ex[0:1] * aw, 0.0, img_size)
    t = jnp.clip(ex[1:2] * ah, 0.0, img_size)
    r = jnp.clip(ex[2:3] * aw, 0.0, img_size)
    b = jnp.clip(ex[3:4] * ah, 0.0, img_size)
    k = jax.lax.broadcasted_iota(jnp.int32, (1, cols), 1)
    p = pl.program_id(1) * cols + k
    gx = (p % 128).astype(jnp.float32)
    gy = (p // 128).astype(jnp.float32)
    cx = (gx + 0.5) * 4.0 + (r - l) * 0.5
    cy = (gy + 0.5) * 4.0 + (b - t) * 0.5
    w = l + r
    h = t + b
    cm = jnp.concatenate([cx, cy, w, h, sg[4:, :]], axis=0)
    out_ref[0] = jax.lax.dot_general(
        cm, eye_ref[...], (((0,), (0,)), ((), ())),
        preferred_element_type=jnp.float32)


def kernel(raw, img_size):
    nB = raw.shape[0]
    nG = raw.shape[2]
    nA = 3
    nCH = _NCH
    cols = _COLS
    inp = raw.reshape(nB * nA, nCH, nG * nG)
    eye = jnp.eye(nCH, dtype=jnp.float32)
    img = jnp.asarray(img_size, dtype=jnp.float32).reshape(1)
    grid = (nB * nA, (nG * nG) // cols)
    out = pl.pallas_call(
        functools.partial(_decode_kernel, cols=cols),
        grid=grid,
        in_specs=[
            pl.BlockSpec(memory_space=pltpu.SMEM),
            pl.BlockSpec((1, nCH, cols), lambda i, j: (i, 0, j)),
            pl.BlockSpec((nCH, nCH), lambda i, j: (0, 0)),
        ],
        out_specs=pl.BlockSpec((1, cols, nCH), lambda i, j: (i, j, 0)),
        out_shape=jax.ShapeDtypeStruct((nB * nA, nG * nG, nCH), jnp.float32),
    )(img, inp, eye)
    return out.reshape(nB, nA * nG * nG, nCH)


# speedup vs baseline: 1.5265x; 1.4060x over previous
import functools

import jax
import jax.numpy as jnp
from jax.experimental import pallas as pl
from jax.experimental.pallas import tpu as pltpu

_NCH = 85
_BY = 32


def _decode_kernel(img_ref, in_ref, eye_ref, out_ref, *, by):
    img_size = img_ref[0]
    a = pl.program_id(1)
    aw = jnp.where(a == 1, 16.0, jnp.where(a == 2, 33.0, 10.0))
    ah = jnp.where(a == 1, 30.0, jnp.where(a == 2, 23.0, 13.0))
    v = in_ref[0]
    sg = 1.0 / (1.0 + jnp.exp(-v))
    ex = jnp.exp(v[0:4])
    l = jnp.clip(ex[0:1] * aw, 0.0, img_size)
    t = jnp.clip(ex[1:2] * ah, 0.0, img_size)
    r = jnp.clip(ex[2:3] * aw, 0.0, img_size)
    b = jnp.clip(ex[3:4] * ah, 0.0, img_size)
    gx = jax.lax.broadcasted_iota(jnp.int32, (1, by, 128), 2).astype(jnp.float32)
    gy = (pl.program_id(2) * by
          + jax.lax.broadcasted_iota(jnp.int32, (1, by, 128), 1)
          ).astype(jnp.float32)
    cx = (gx + 0.5) * 4.0 + (r - l) * 0.5
    cy = (gy + 0.5) * 4.0 + (b - t) * 0.5
    w = l + r
    h = t + b
    cm = jnp.concatenate([cx, cy, w, h, sg[4:]], axis=0)
    out_ref[0, 0] = jax.lax.dot_general(
        cm, eye_ref[...], (((0,), (0,)), ((), ())),
        preferred_element_type=jnp.float32)


def kernel(raw, img_size):
    nB = raw.shape[0]
    nG = raw.shape[2]
    nA = 3
    nCH = _NCH
    by = _BY
    eye = jnp.eye(nCH, dtype=jnp.float32)
    img = jnp.asarray(img_size, dtype=jnp.float32).reshape(1)
    grid = (nB, nA, nG // by)
    out = pl.pallas_call(
        functools.partial(_decode_kernel, by=by),
        grid=grid,
        in_specs=[
            pl.BlockSpec(memory_space=pltpu.SMEM),
            pl.BlockSpec((1, nCH, by, 128), lambda b, a, j: (b, a, j, 0)),
            pl.BlockSpec((nCH, nCH), lambda b, a, j: (0, 0)),
        ],
        out_specs=pl.BlockSpec((1, 1, by, 128, nCH),
                               lambda b, a, j: (b, a, j, 0, 0)),
        out_shape=jax.ShapeDtypeStruct((nB, nA, nG, nG, nCH), jnp.float32),
    )(img, raw, eye)
    return out.reshape(nB, nA * nG * nG, nCH)


# speedup vs baseline: 1.8128x; 1.1876x over previous
import functools

import jax
import jax.numpy as jnp
from jax.experimental import pallas as pl
from jax.experimental.pallas import tpu as pltpu

_NCH = 85
_BY = 128


def _decode_kernel(img_ref, in_ref, eye_ref, out_ref, *, by):
    img_size = img_ref[0]
    a = pl.program_id(1)
    aw = jnp.where(a == 1, 16.0, jnp.where(a == 2, 33.0, 10.0))
    ah = jnp.where(a == 1, 30.0, jnp.where(a == 2, 23.0, 13.0))
    v = in_ref[0]
    sg = 1.0 / (1.0 + jnp.exp(-v))
    ex = jnp.exp(v[0:4])
    l = jnp.clip(ex[0:1] * aw, 0.0, img_size)
    t = jnp.clip(ex[1:2] * ah, 0.0, img_size)
    r = jnp.clip(ex[2:3] * aw, 0.0, img_size)
    b = jnp.clip(ex[3:4] * ah, 0.0, img_size)
    gx = jax.lax.broadcasted_iota(jnp.int32, (1, by, 128), 2).astype(jnp.float32)
    gy = (pl.program_id(2) * by
          + jax.lax.broadcasted_iota(jnp.int32, (1, by, 128), 1)
          ).astype(jnp.float32)
    cx = (gx + 0.5) * 4.0 + (r - l) * 0.5
    cy = (gy + 0.5) * 4.0 + (b - t) * 0.5
    w = l + r
    h = t + b
    cm = jnp.concatenate([cx, cy, w, h, sg[4:]], axis=0)
    out_ref[0, 0] = jax.lax.dot_general(
        cm, eye_ref[...], (((0,), (0,)), ((), ())),
        preferred_element_type=jnp.float32)


def kernel(raw, img_size):
    nB = raw.shape[0]
    nG = raw.shape[2]
    nA = 3
    nCH = _NCH
    by = _BY
    eye = jnp.eye(nCH, dtype=jnp.float32)
    img = jnp.asarray(img_size, dtype=jnp.float32).reshape(1)
    grid = (nB, nA, nG // by)
    out = pl.pallas_call(
        functools.partial(_decode_kernel, by=by),
        grid=grid,
        in_specs=[
            pl.BlockSpec(memory_space=pltpu.SMEM),
            pl.BlockSpec((1, nCH, by, 128), lambda b, a, j: (b, a, j, 0)),
            pl.BlockSpec((nCH, nCH), lambda b, a, j: (0, 0)),
        ],
        out_specs=pl.BlockSpec((1, 1, by, 128, nCH),
                               lambda b, a, j: (b, a, j, 0, 0)),
        out_shape=jax.ShapeDtypeStruct((nB, nA, nG, nG, nCH), jnp.float32),
    )(img, raw, eye)
    return out.reshape(nB, nA * nG * nG, nCH)


# speedup vs baseline: 2.1699x; 1.1970x over previous
import jax
import jax.numpy as jnp
from jax.experimental import pallas as pl
from jax.experimental.pallas import tpu as pltpu

_NCH = 85


def _decode_kernel(img_ref, in_ref, out_ref):
    img_size = img_ref[0]
    a = pl.program_id(1)
    aw = jnp.where(a == 1, 16.0, jnp.where(a == 2, 33.0, 10.0))
    ah = jnp.where(a == 1, 30.0, jnp.where(a == 2, 23.0, 13.0))
    v = in_ref[0]
    sg = 1.0 / (1.0 + jnp.exp(-v))
    ex = jnp.exp(v[0:4])
    l = jnp.clip(ex[0:1] * aw, 0.0, img_size)
    t = jnp.clip(ex[1:2] * ah, 0.0, img_size)
    r = jnp.clip(ex[2:3] * aw, 0.0, img_size)
    b = jnp.clip(ex[3:4] * ah, 0.0, img_size)
    gx = jax.lax.broadcasted_iota(jnp.int32, (1, 128, 128), 2).astype(jnp.float32)
    gy = jax.lax.broadcasted_iota(jnp.int32, (1, 128, 128), 1).astype(jnp.float32)
    cx = (gx + 0.5) * 4.0 + (r - l) * 0.5
    cy = (gy + 0.5) * 4.0 + (b - t) * 0.5
    w = l + r
    h = t + b
    out_ref[:, 0, 0] = jnp.concatenate([cx, cy, w, h, sg[4:]], axis=0)


def kernel(raw, img_size):
    nB = raw.shape[0]
    nG = raw.shape[2]
    nA = 3
    nCH = _NCH
    img = jnp.asarray(img_size, dtype=jnp.float32).reshape(1)
    grid = (nB, nA)
    out = pl.pallas_call(
        _decode_kernel,
        grid=grid,
        in_specs=[
            pl.BlockSpec(memory_space=pltpu.SMEM),
            pl.BlockSpec((1, nCH, nG, nG), lambda b, a: (b, a, 0, 0)),
        ],
        out_specs=pl.BlockSpec((nCH, 1, 1, nG, nG),
                               lambda b, a: (0, b, a, 0, 0)),
        out_shape=jax.ShapeDtypeStruct((nCH, nB, nA, nG, nG), jnp.float32),
    )(img, raw)
    return out.transpose(1, 2, 3, 4, 0).reshape(nB, nA * nG * nG, nCH)


# speedup vs baseline: 4.7816x; 2.2036x over previous
import jax
import jax.numpy as jnp
from jax.experimental import pallas as pl
from jax.experimental.pallas import tpu as pltpu

_NCH = 85
_CG = 5


def _decode_kernel(img_ref, in_ref, out_ref):
    img_size = img_ref[0]
    g = pl.program_id(0)
    a = pl.program_id(1)
    aw = jnp.where(a == 1, 16.0, jnp.where(a == 2, 33.0, 10.0))
    ah = jnp.where(a == 1, 30.0, jnp.where(a == 2, 23.0, 13.0))
    v = in_ref[...]
    nb, _, ng, _ = v.shape
    sig = 1.0 / (1.0 + jnp.exp(-v))
    out_ref[...] = jnp.transpose(sig, (1, 0, 2, 3)).reshape(_CG, nb, ng * ng)

    @pl.when(g == 0)
    def _decode_boxes():
        ex = jnp.exp(v[:, 0:4])
        anc = jnp.where(
            jax.lax.broadcasted_iota(jnp.int32, (1, 4, 1, 1), 1) % 2 == 0,
            aw, ah)
        e = jnp.clip(ex * anc, 0.0, img_size)
        l = e[:, 0:1]
        t = e[:, 1:2]
        r = e[:, 2:3]
        b = e[:, 3:4]
        gx = jax.lax.broadcasted_iota(
            jnp.int32, (1, 1, ng, ng), 3).astype(jnp.float32)
        gy = jax.lax.broadcasted_iota(
            jnp.int32, (1, 1, ng, ng), 2).astype(jnp.float32)
        cx = (gx + 0.5) * 4.0 + (r - l) * 0.5
        cy = (gy + 0.5) * 4.0 + (b - t) * 0.5
        w = l + r
        h = t + b
        xy = jnp.concatenate([cx, cy, w, h], axis=1)
        out_ref[0:4] = jnp.transpose(xy, (1, 0, 2, 3)).reshape(4, nb, ng * ng)


def kernel(raw, img_size):
    nB = raw.shape[0]
    nG = raw.shape[2]
    nA = 3
    nCH = _NCH
    img = jnp.asarray(img_size, dtype=jnp.float32).reshape(1)
    ngrp = nCH // _CG
    grid = (ngrp, nA)
    out = pl.pallas_call(
        _decode_kernel,
        grid=grid,
        in_specs=[
            pl.BlockSpec(memory_space=pltpu.SMEM),
            pl.BlockSpec((nB, _CG, nG, nG), lambda g, a: (0, ngrp * a + g, 0, 0)),
        ],
        out_specs=pl.BlockSpec((_CG, nB, nG * nG), lambda g, a: (g, 0, a)),
        out_shape=jax.ShapeDtypeStruct((nCH, nB, nA * nG * nG), jnp.float32),
    )(img, raw)
    return out.transpose(1, 2, 0)


# speedup vs baseline: 4.8963x; 1.0240x over previous
import jax
import jax.numpy as jnp
from jax.experimental import pallas as pl
from jax.experimental.pallas import tpu as pltpu

_NCH = 85
_CG = 5


def _decode_kernel(img_ref, in_ref, out_ref):
    img_size = img_ref[0]
    g = pl.program_id(0)
    a = pl.program_id(1)
    aw = jnp.where(a == 1, 16.0, jnp.where(a == 2, 33.0, 10.0))
    ah = jnp.where(a == 1, 30.0, jnp.where(a == 2, 23.0, 13.0))
    v = in_ref[...]
    nb, _, ng, _ = v.shape
    sig = 0.5 * jnp.tanh(0.5 * v) + 0.5
    out_ref[...] = jnp.transpose(sig, (1, 0, 2, 3)).reshape(_CG, nb, ng * ng)

    @pl.when(g == 0)
    def _decode_boxes():
        ex = jnp.exp(v[:, 0:4])
        anc = jnp.where(
            jax.lax.broadcasted_iota(jnp.int32, (1, 4, 1, 1), 1) % 2 == 0,
            aw, ah)
        e = jnp.clip(ex * anc, 0.0, img_size)
        l = e[:, 0:1]
        t = e[:, 1:2]
        r = e[:, 2:3]
        b = e[:, 3:4]
        gx = jax.lax.broadcasted_iota(
            jnp.int32, (1, 1, ng, ng), 3).astype(jnp.float32)
        gy = jax.lax.broadcasted_iota(
            jnp.int32, (1, 1, ng, ng), 2).astype(jnp.float32)
        cx = (gx + 0.5) * 4.0 + (r - l) * 0.5
        cy = (gy + 0.5) * 4.0 + (b - t) * 0.5
        w = l + r
        h = t + b
        xy = jnp.concatenate([cx, cy, w, h], axis=1)
        out_ref[0:4] = jnp.transpose(xy, (1, 0, 2, 3)).reshape(4, nb, ng * ng)


def kernel(raw, img_size):
    nB = raw.shape[0]
    nG = raw.shape[2]
    nA = 3
    nCH = _NCH
    img = jnp.asarray(img_size, dtype=jnp.float32).reshape(1)
    ngrp = nCH // _CG
    grid = (ngrp, nA)
    out = pl.pallas_call(
        _decode_kernel,
        grid=grid,
        in_specs=[
            pl.BlockSpec(memory_space=pltpu.SMEM),
            pl.BlockSpec((nB, _CG, nG, nG), lambda g, a: (0, ngrp * a + g, 0, 0)),
        ],
        out_specs=pl.BlockSpec((_CG, nB, nG * nG), lambda g, a: (g, 0, a)),
        out_shape=jax.ShapeDtypeStruct((nCH, nB, nA * nG * nG), jnp.float32),
    )(img, raw)
    return out.transpose(1, 2, 0)
